# baseline (speedup 1.0000x reference)
import jax
import jax.numpy as jnp
from jax.experimental import pallas as pl
from jax.experimental.pallas import tpu as pltpu

_TOKENS = 32768
_DIM = 768
_EXPERTS = 64
_BT = 4096
_BD = 256
_NG = _TOKENS // _BT
_NC = _DIM // _BD


def _router_kernel(xw_ref, cw_ref, cfull_ref, t_ref, out_ref, ss_ref, cninv_ref):
    c = pl.program_id(1)

    @pl.when((pl.program_id(0) == 0) & (c == 0))
    def _init_cn():
        cf = cfull_ref[:]
        c_ss = jnp.sum(cf * cf, axis=1, keepdims=True)
        cninv_ref[:] = jax.lax.rsqrt(jnp.maximum(c_ss, 1e-24))

    xb = xw_ref[:]
    cnw = cw_ref[:] * cninv_ref[:]
    ssp = jnp.sum(xb * xb, axis=1, keepdims=True)
    partial = jax.lax.dot_general(
        xb, cnw, (((1,), (1,)), ((), ())), preferred_element_type=jnp.float32
    )

    @pl.when(c == 0)
    def _first():
        out_ref[:] = partial
        ss_ref[:] = ssp

    @pl.when(c > 0)
    def _acc():
        out_ref[:] += partial
        ss_ref[:] += ssp

    @pl.when(c == _NC - 1)
    def _final():
        inv_norm = jax.lax.rsqrt(jnp.maximum(ss_ref[:], 1e-24))
        out_ref[:] = out_ref[:] * (inv_norm / t_ref[0])


@jax.jit
def kernel(x, centroids, temperature):
    return pl.pallas_call(
        _router_kernel,
        grid=(_NG, _NC),
        in_specs=[
            pl.BlockSpec((_BT, _BD), lambda g, c: (g, c)),
            pl.BlockSpec((_EXPERTS, _BD), lambda g, c: (0, c)),
            pl.BlockSpec((_EXPERTS, _DIM), lambda g, c: (0, 0)),
            pl.BlockSpec(memory_space=pltpu.SMEM),
        ],
        out_specs=pl.BlockSpec((_BT, _EXPERTS), lambda g, c: (g, 0)),
        out_shape=jax.ShapeDtypeStruct((_TOKENS, _EXPERTS), jnp.float32),
        scratch_shapes=[
            pltpu.VMEM((_BT, 1), jnp.float32),
            pltpu.VMEM((_EXPERTS, 1), jnp.float32),
        ],
        compiler_params=pltpu.CompilerParams(
            dimension_semantics=("arbitrary", "arbitrary"),
        ),
    )(x, centroids, centroids, temperature)
